# Initial kernel scaffold; baseline (speedup 1.0000x reference)
import jax
import jax.numpy as jnp
from jax.experimental import pallas as pl


def kernel(boxes, scores):
    raise NotImplementedError("write your pallas kernel here")


# speedup vs baseline: 149.1772x; 149.1772x over previous
import jax
import jax.numpy as jnp
from jax import lax
from jax.experimental import pallas as pl
from jax.experimental.pallas import tpu as pltpu

_NB = 20000
_PRE = 12000
_POST = 2000
_THR = 0.7
_MIN = 16.0
_IMW = 800.0
_IMH = 800.0

_L = 128
_R = 96
_NPAD = _R * _L
_RPT = 8
_B = _RPT * _L
_T = _R // _RPT

_HI = lax.Precision.HIGHEST


def _nms_kernel(x1_ref, y1_ref, x2_ref, y2_ref, keep_out,
                keep_s, area_s, ident_s, m_s):
    i = pl.program_id(0)

    @pl.when(i == 0)
    def _init():
        fi = (lax.broadcasted_iota(jnp.int32, (_R, _L), 0) * _L
              + lax.broadcasted_iota(jnp.int32, (_R, _L), 1))
        keep_s[:] = (fi < _PRE).astype(jnp.float32)
        area_s[:] = (x2_ref[:] - x1_ref[:] + 1.0) * (y2_ref[:] - y1_ref[:] + 1.0)
        ident_s[:] = (lax.broadcasted_iota(jnp.int32, (_L, _L), 0)
                      == lax.broadcasted_iota(jnp.int32, (_L, _L), 1)
                      ).astype(jnp.float32)

    def _t_col(row):
        return lax.dot_general(ident_s[:], row, (((1,), (1,)), ((), ())),
                               preferred_element_type=jnp.float32, precision=_HI)

    def _t_row(col):
        return lax.dot_general(col, ident_s[:], (((0,), (0,)), ((), ())),
                               preferred_element_type=jnp.float32, precision=_HI)

    r0 = i * _RPT

    def _cols(ref):
        return jnp.concatenate(
            [_t_col(ref[pl.ds(r0 + rr, 1), :]) for rr in range(_RPT)], axis=0)

    tx1 = _cols(x1_ref)
    ty1 = _cols(y1_ref)
    tx2 = _cols(x2_ref)
    ty2 = _cols(y2_ref)
    tarea = _cols(area_s)
    talive = _cols(keep_s)
    trow = lax.broadcasted_iota(jnp.int32, (_B, 1), 0)

    def _ovr(cx1, cy1, cx2, cy2, carea):
        w = jnp.maximum(0.0, jnp.minimum(tx2, cx2) - jnp.maximum(tx1, cx1) + 1.0)
        h = jnp.maximum(0.0, jnp.minimum(ty2, cy2) - jnp.maximum(ty1, cy1) + 1.0)
        inter = w * h
        return inter / (tarea + carea - inter)

    for rr in range(_RPT):
        row = r0 + rr
        ovr = _ovr(x1_ref[pl.ds(row, 1), :], y1_ref[pl.ds(row, 1), :],
                   x2_ref[pl.ds(row, 1), :], y2_ref[pl.ds(row, 1), :],
                   area_s[pl.ds(row, 1), :])
        colj = rr * _L + lax.broadcasted_iota(jnp.int32, (1, _L), 1)
        flag = (ovr > _THR) & (colj > trow)
        m_s[:, pl.ds(rr * _L, _L)] = flag.astype(jnp.float32)

    def _cond(c):
        return c[1]

    def _body(c):
        k, _ = c
        sup = lax.dot_general(m_s[:], k, (((0,), (0,)), ((), ())),
                              preferred_element_type=jnp.float32, precision=_HI)
        nk = jnp.where(sup > 0.5, 0.0, talive)
        return nk, jnp.sum(jnp.abs(nk - k)) > 0.0

    keep_t, _ = lax.while_loop(_cond, _body, (talive, True))
    for rr in range(_RPT):
        keep_s[pl.ds(r0 + rr, 1), :] = _t_row(keep_t[rr * _L:(rr + 1) * _L, :])

    def _cross(r, carry):
        ovr = _ovr(x1_ref[pl.ds(r, 1), :], y1_ref[pl.ds(r, 1), :],
                   x2_ref[pl.ds(r, 1), :], y2_ref[pl.ds(r, 1), :],
                   area_s[pl.ds(r, 1), :])
        flag = (ovr > _THR).astype(jnp.float32)
        sup = lax.dot_general(keep_t, flag, (((0,), (0,)), ((), ())),
                              preferred_element_type=jnp.float32, precision=_HI)
        keep_s[pl.ds(r, 1), :] = jnp.where(sup > 0.5, 0.0, keep_s[pl.ds(r, 1), :])
        return carry

    lax.fori_loop(r0 + _RPT, _R, _cross, 0)

    @pl.when(i == _T - 1)
    def _fin():
        keep_out[:] = keep_s[:]


def _nms_keep(x1, y1, x2, y2):
    return pl.pallas_call(
        _nms_kernel,
        grid=(_T,),
        in_specs=[pl.BlockSpec((_R, _L), lambda i: (0, 0))] * 4,
        out_specs=pl.BlockSpec((_R, _L), lambda i: (0, 0)),
        out_shape=jax.ShapeDtypeStruct((_R, _L), jnp.float32),
        scratch_shapes=[
            pltpu.VMEM((_R, _L), jnp.float32),
            pltpu.VMEM((_R, _L), jnp.float32),
            pltpu.VMEM((_L, _L), jnp.float32),
            pltpu.VMEM((_B, _B), jnp.float32),
        ],
    )(x1, y1, x2, y2)


def kernel(boxes, scores):
    x1 = jnp.clip(boxes[:, 0], 0.0, _IMW - 1.0)
    y1 = jnp.clip(boxes[:, 1], 0.0, _IMH - 1.0)
    x2 = jnp.clip(boxes[:, 2], 0.0, _IMW - 1.0)
    y2 = jnp.clip(boxes[:, 3], 0.0, _IMH - 1.0)
    ws = x2 - x1 + 1.0
    hs = y2 - y1 + 1.0
    size_ok = (ws >= _MIN) & (hs >= _MIN)
    sc = jnp.where(size_ok, scores, -jnp.inf)

    _, x1s, y1s, x2s, y2s, scs = lax.sort(
        (-sc, x1, y1, x2, y2, sc), dimension=0, num_keys=1, is_stable=True)

    pad = _NPAD - _PRE

    def _prep(a):
        return jnp.concatenate([a[:_PRE], jnp.zeros((pad,), a.dtype)]).reshape(_R, _L)

    keep = _nms_keep(_prep(x1s), _prep(y1s), _prep(x2s), _prep(y2s))

    keepf = keep.reshape(-1)[:_PRE] > 0.5
    rank = jnp.cumsum(keepf.astype(jnp.int32)) - 1
    valid = keepf & (rank < _POST)
    idx = jnp.where(valid, rank, _POST)
    props = jnp.stack([x1s[:_PRE], y1s[:_PRE], x2s[:_PRE], y2s[:_PRE]], axis=1)
    psc = scs[:_PRE]
    out_b = jnp.zeros((_POST, 4), props.dtype).at[idx].set(props, mode='drop')
    out_s = jnp.zeros((_POST,), psc.dtype).at[idx].set(psc, mode='drop')
    return jnp.concatenate([out_b, out_s[:, None]], axis=1)


# speedup vs baseline: 443.8958x; 2.9756x over previous
import jax
import jax.numpy as jnp
from jax import lax
from jax.experimental import pallas as pl
from jax.experimental.pallas import tpu as pltpu

_NB = 20000
_PRE = 12000
_POST = 2000
_THR = 0.7
_MIN = 16.0
_IMW = 800.0
_IMH = 800.0

_B = 1024
_T = 12
_NPAD = _T * _B

_HI = lax.Precision.HIGHEST


def _nms_kernel(x1_ref, y1_ref, x2_ref, y2_ref, keep_out,
                ident_s, m_s, area_s, cnt_s):
    i = pl.program_id(0)

    @pl.when(i == 0)
    def _init():
        ident_s[:] = (lax.broadcasted_iota(jnp.int32, (_B, _B), 0)
                      == lax.broadcasted_iota(jnp.int32, (_B, _B), 1)
                      ).astype(jnp.float32)
        area_s[:] = (x2_ref[:] - x1_ref[:] + 1.0) * (y2_ref[:] - y1_ref[:] + 1.0)
        cnt_s[0] = 0.0

    done = cnt_s[0] >= float(_POST)

    @pl.when(jnp.logical_not(done))
    def _tile():
        def _t_col(row):
            return lax.dot_general(ident_s[:], row, (((1,), (1,)), ((), ())),
                                   preferred_element_type=jnp.float32,
                                   precision=_HI)

        cx1 = _t_col(x1_ref[pl.ds(i, 1), :])
        cy1 = _t_col(y1_ref[pl.ds(i, 1), :])
        cx2 = _t_col(x2_ref[pl.ds(i, 1), :])
        cy2 = _t_col(y2_ref[pl.ds(i, 1), :])
        carea = (cx2 - cx1 + 1.0) * (cy2 - cy1 + 1.0)
        jidx = lax.broadcasted_iota(jnp.int32, (_B, 1), 0)
        galive = ((i * _B + jidx) < _PRE).astype(jnp.float32)

        def _ovr_row(t):
            rx1 = x1_ref[pl.ds(t, 1), :]
            ry1 = y1_ref[pl.ds(t, 1), :]
            rx2 = x2_ref[pl.ds(t, 1), :]
            ry2 = y2_ref[pl.ds(t, 1), :]
            rarea = area_s[pl.ds(t, 1), :]
            w = jnp.maximum(0.0, jnp.minimum(cx2, rx2) - jnp.maximum(cx1, rx1) + 1.0)
            h = jnp.maximum(0.0, jnp.minimum(cy2, ry2) - jnp.maximum(cy1, ry1) + 1.0)
            inter = w * h
            return inter / (carea + rarea - inter)

        def _tbody(t, sup):
            flag = (_ovr_row(t) > _THR).astype(jnp.float32)
            krow = keep_out[pl.ds(t, 1), :]
            return sup + jnp.sum(flag * krow, axis=1, keepdims=True)

        sup0 = lax.fori_loop(0, i, _tbody, jnp.zeros((_B, 1), jnp.float32))
        alive = jnp.where(sup0 > 0.5, 0.0, galive)

        kidx = lax.broadcasted_iota(jnp.int32, (1, _B), 1)
        m_s[:] = ((_ovr_row(i) > _THR) & (jidx > kidx)).astype(jnp.float32)

        def _cond(c):
            return c[1]

        def _body(c):
            k, _ = c
            sup = lax.dot_general(m_s[:], k, (((1,), (0,)), ((), ())),
                                  preferred_element_type=jnp.float32,
                                  precision=_HI)
            nk = jnp.where(sup > 0.5, 0.0, alive)
            return nk, jnp.sum(jnp.abs(nk - k)) > 0.0

        keep_t, _ = lax.while_loop(_cond, _body, (alive, True))

        keep_out[pl.ds(i, 1), :] = lax.dot_general(
            keep_t, ident_s[:], (((0,), (0,)), ((), ())),
            preferred_element_type=jnp.float32, precision=_HI)
        cnt_s[0] = cnt_s[0] + jnp.sum(keep_t)

    @pl.when(done)
    def _skip():
        lane = lax.broadcasted_iota(jnp.int32, (1, _B), 1)
        keep_out[pl.ds(i, 1), :] = ((i * _B + lane) < _PRE).astype(jnp.float32)


def _nms_keep(x1, y1, x2, y2):
    return pl.pallas_call(
        _nms_kernel,
        grid=(_T,),
        in_specs=[pl.BlockSpec((_T, _B), lambda i: (0, 0))] * 4,
        out_specs=pl.BlockSpec((_T, _B), lambda i: (0, 0)),
        out_shape=jax.ShapeDtypeStruct((_T, _B), jnp.float32),
        scratch_shapes=[
            pltpu.VMEM((_B, _B), jnp.float32),
            pltpu.VMEM((_B, _B), jnp.float32),
            pltpu.VMEM((_T, _B), jnp.float32),
            pltpu.SMEM((1,), jnp.float32),
        ],
    )(x1, y1, x2, y2)


def kernel(boxes, scores):
    x1 = jnp.clip(boxes[:, 0], 0.0, _IMW - 1.0)
    y1 = jnp.clip(boxes[:, 1], 0.0, _IMH - 1.0)
    x2 = jnp.clip(boxes[:, 2], 0.0, _IMW - 1.0)
    y2 = jnp.clip(boxes[:, 3], 0.0, _IMH - 1.0)
    ws = x2 - x1 + 1.0
    hs = y2 - y1 + 1.0
    size_ok = (ws >= _MIN) & (hs >= _MIN)
    sc = jnp.where(size_ok, scores, -jnp.inf)

    _, x1s, y1s, x2s, y2s, scs = lax.sort(
        (-sc, x1, y1, x2, y2, sc), dimension=0, num_keys=1, is_stable=True)

    pad = _NPAD - _PRE

    def _prep(a):
        return jnp.concatenate([a[:_PRE], jnp.zeros((pad,), a.dtype)]).reshape(_T, _B)

    keep = _nms_keep(_prep(x1s), _prep(y1s), _prep(x2s), _prep(y2s))

    keepf = keep.reshape(-1)[:_PRE] > 0.5
    rank = jnp.cumsum(keepf.astype(jnp.int32)) - 1
    valid = keepf & (rank < _POST)
    idx = jnp.where(valid, rank, _POST)
    props = jnp.stack([x1s[:_PRE], y1s[:_PRE], x2s[:_PRE], y2s[:_PRE]], axis=1)
    psc = scs[:_PRE]
    out_b = jnp.zeros((_POST, 4), props.dtype).at[idx].set(props, mode='drop')
    out_s = jnp.zeros((_POST,), psc.dtype).at[idx].set(psc, mode='drop')
    return jnp.concatenate([out_b, out_s[:, None]], axis=1)
